# baseline (speedup 1.0000x reference)
import functools

import jax
import jax.numpy as jnp
from jax import lax
from jax.experimental import pallas as pl
from jax.experimental.pallas import tpu as pltpu
from jax.experimental.pallas import tpu_sc as plsc

K = 20
EPS = 1e-5
NEG_INF = float("-inf")


def _a_body(xb_ref, xr_ref, w1t_ref, wst_ref, p_ref, zt_ref, yt_ref):
    xb = xb_ref[0]
    xr = xr_ref[0]
    dn = (((0,), (0,)), ((), ()))
    g = lax.dot_general(xr, xb, dn, preferred_element_type=jnp.float32)
    xx_col = jnp.sum(xb * xb, axis=0, keepdims=True)
    xx_row = jnp.sum(xr * xr, axis=0)[:, None]
    inner = -2.0 * g
    p_ref[0] = (-xx_col - inner) - xx_row
    zt_ref[0] = lax.dot_general(xr, w1t_ref[...], dn,
                                preferred_element_type=jnp.float32)
    yt_ref[0] = lax.dot_general(xr, wst_ref[...], dn,
                                preferred_element_type=jnp.float32)


def _pairwise_and_tables(x, w1t, wst, rb=256):
    b, c, n = x.shape
    o = w1t.shape[1]
    nrb = n // rb
    return pl.pallas_call(
        _a_body,
        grid=(b, nrb),
        in_specs=[
            pl.BlockSpec((1, c, n), lambda i, j: (i, 0, 0)),
            pl.BlockSpec((1, c, rb), lambda i, j: (i, 0, j)),
            pl.BlockSpec((c, o), lambda i, j: (0, 0)),
            pl.BlockSpec((c, o), lambda i, j: (0, 0)),
        ],
        out_specs=[
            pl.BlockSpec((1, rb, n), lambda i, j: (i, j, 0)),
            pl.BlockSpec((1, rb, o), lambda i, j: (i, j, 0)),
            pl.BlockSpec((1, rb, o), lambda i, j: (i, j, 0)),
        ],
        out_shape=[
            jax.ShapeDtypeStruct((b, n, n), jnp.float32),
            jax.ShapeDtypeStruct((b, n, o), jnp.float32),
            jax.ShapeDtypeStruct((b, n, o), jnp.float32),
        ],
    )(x, x, w1t, wst)


def _b_body(p_ref, idx_ref):
    bidx = pl.program_id(0)
    p = p_ref[0]
    r, n = p.shape
    iota = lax.broadcasted_iota(jnp.int32, (r, n), 1)
    tiota = lax.broadcasted_iota(jnp.int32, (r, 32), 1)
    off = bidx * n
    acc = jnp.zeros((r, 32), jnp.int32)
    for t in range(K):
        m = jnp.max(p, axis=1, keepdims=True)
        cand = jnp.where(p == m, iota, n)
        j = jnp.min(cand, axis=1, keepdims=True)
        acc = jnp.where(tiota == t, j + off, acc)
        p = jnp.where(iota == j, NEG_INF, p)
    idx_ref[0] = acc


def _topk_tc(p, rb=512):
    b, n, _ = p.shape
    return pl.pallas_call(
        _b_body,
        grid=(b, n // rb),
        in_specs=[pl.BlockSpec((1, rb, n), lambda i, j: (i, j, 0))],
        out_specs=pl.BlockSpec((1, rb, 32), lambda i, j: (i, j, 0)),
        out_shape=jax.ShapeDtypeStruct((b, n, 32), jnp.int32),
    )(p)


def _topk_sc(p_flat):
    nt, n = p_flat.shape
    info = plsc.get_sparse_core_info()
    L = info.num_lanes
    nw = info.num_cores * info.num_subcores
    rpw = nt // nw
    nv = n // L
    rpb = 4
    nbuf = 2
    och = 64
    nslots = rpw // rpb
    spf = och // rpb
    mesh = plsc.VectorSubcoreMesh(core_axis_name="c", subcore_axis_name="s")

    cap = 112

    @functools.partial(
        pl.kernel,
        mesh=mesh,
        out_type=jax.ShapeDtypeStruct((nt, 32), jnp.int32),
        compiler_params=pltpu.CompilerParams(needs_layout_passes=False),
        scratch_types=[
            pltpu.VMEM((nbuf, rpb, n), jnp.float32),
            pltpu.VMEM((och, 32), jnp.int32),
            pltpu.VMEM((cap + L,), jnp.float32),
            pltpu.VMEM((cap + L,), jnp.int32),
            pltpu.SemaphoreType.DMA,
            pltpu.SemaphoreType.DMA,
        ],
    )
    def tkern(p_hbm, idx_hbm, ring_v, ob, cand_v, cand_i, sem_a, sem_b):
        wid = lax.axis_index("s") * info.num_cores + lax.axis_index("c")
        r0 = wid * rpw
        cbase = (r0 // n) * n
        sems = (sem_a, sem_b)
        iota = lax.iota(jnp.int32, L)

        def issue(slot, bb):
            pltpu.async_copy(p_hbm.at[pl.ds(r0 + slot * rpb, rpb)],
                             ring_v.at[bb], sems[bb])

        def drain(bb):
            pltpu.make_async_copy(p_hbm.at[pl.ds(0, rpb)], ring_v.at[bb],
                                  sems[bb]).wait()

        def bsplit(k0, v0, k1, v1):
            mx = k0 >= k1
            hk = jnp.where(mx, k0, k1)
            hv = jnp.where(mx, v0, v1)
            lk = jnp.where(mx, k1, k0)
            lv = jnp.where(mx, v1, v0)
            hk, hv = plsc.sort_key_val(hk, hv, descending=True)
            lk, lv = plsc.sort_key_val(lk, lv, descending=True)
            return ((hk, hv), (lk, lv))

        def leaf(k0, v0, k1, v1):
            k0, v0 = plsc.sort_key_val(k0, v0, descending=True)
            k1, v1 = plsc.sort_key_val(k1, v1, descending=False)
            return bsplit(k0, v0, k1, v1)

        def merge32(a, b):
            (ak0, av0), (ak1, av1) = a
            (bk0, bv0), (bk1, bv1) = b
            rk0 = lax.rev(bk1, (0,))
            rv0 = lax.rev(bv1, (0,))
            rk1 = lax.rev(bk0, (0,))
            rv1 = lax.rev(bv0, (0,))
            m0 = ak0 >= rk0
            c0k = jnp.where(m0, ak0, rk0)
            c0v = jnp.where(m0, av0, rv0)
            m1 = ak1 >= rk1
            c1k = jnp.where(m1, ak1, rk1)
            c1v = jnp.where(m1, av1, rv1)
            return bsplit(c0k, c0v, c1k, c1v)

        ninf = jnp.full((L,), NEG_INF, jnp.float32)
        zero = jnp.zeros((L,), jnp.int32)

        def store_run(run, rl):
            (_, hv), (_, lv) = run
            ob[rl, pl.ds(0, L)] = hv + cbase
            ob[rl, pl.ds(L, L)] = lv + cbase

        def tree8(loadk, loadv, base_v):
            nodes = []
            for e in range(4):
                v2 = base_v + 2 * e
                nodes.append(leaf(loadk(v2), loadv(v2),
                                  loadk(v2 + 1), loadv(v2 + 1)))
            while len(nodes) > 1:
                nodes = [merge32(nodes[i], nodes[i + 1])
                         for i in range(0, len(nodes), 2)]
            return nodes[0]

        def row_topk_slow(bb, ri, rl):
            def load(v):
                return ring_v[bb, ri, pl.ds(v * L, L)]

            def gbody(g, run):
                node = tree8(load, lambda v: iota + v * L, g * 8)
                return merge32(run, node)

            run0 = ((ninf, zero), (ninf, zero))
            run = lax.fori_loop(0, nv // 8, gbody, run0)
            store_run(run, rl)
            return 0

        def row_topk_fast(bb, ri, rl, t):
            def load(v):
                return ring_v[bb, ri, pl.ds(v * L, L)]

            for j in range((cap + L) // L):
                cand_v[pl.ds(j * L, L)] = ninf

            def p2(v, off_vec):
                x = load(v)
                msk = x >= t
                offc = jnp.minimum(jnp.max(off_vec), cap)
                plsc.store_compressed(cand_v.at[pl.ds(offc, L)], x, mask=msk)
                plsc.store_compressed(cand_i.at[pl.ds(offc, L)],
                                      iota + v * L, mask=msk)
                return off_vec + plsc.all_reduce_population_count(msk)

            off_vec = lax.fori_loop(0, nv, p2, jnp.zeros((L,), jnp.int32),
                                    unroll=8)
            off = jnp.max(off_vec)

            @pl.when(off <= cap)
            def _():
                run = tree8(lambda v: cand_v[pl.ds(v * L, L)],
                            lambda v: cand_i[pl.ds(v * L, L)], 0)
                store_run(run, rl)

            @pl.when(off > cap)
            def _():
                row_topk_slow(bb, ri, rl)

        def row_topk(bb, ri, rl):
            def load(v):
                return ring_v[bb, ri, pl.ds(v * L, L)]

            def p1(v, carry):
                m1, m2 = carry
                x = load(v)
                m2 = jnp.maximum(m2, jnp.minimum(m1, x))
                m1 = jnp.maximum(m1, x)
                return m1, m2

            m1, m2 = lax.fori_loop(1, nv, p1, (load(0), ninf), unroll=8)
            (_, _), (lk, _) = leaf(m1, zero, m2, zero)
            t = jnp.max(jnp.where(iota == 3, lk, NEG_INF))
            row_topk_fast(bb, ri, rl, t)

        for bb in range(nbuf):
            issue(bb, bb)

        def flush_body(fc, _):
            def pair_body(sp, _):
                for bb in range(nbuf):
                    si = sp * nbuf + bb
                    slot = fc * spf + si
                    drain(bb)

                    def rbody(ri, _):
                        row_topk(bb, ri, si * rpb + ri)
                        return 0

                    lax.fori_loop(0, rpb, rbody, 0)

                    @pl.when(slot + nbuf < nslots)
                    def _():
                        issue(slot + nbuf, bb)
                return 0

            lax.fori_loop(0, spf // nbuf, pair_body, 0)
            pltpu.sync_copy(ob, idx_hbm.at[pl.ds(r0 + fc * och, och)])
            return 0

        lax.fori_loop(0, nslots // spf, flush_body, 0)

    return tkern(p_flat)


def _gather_reduce_sc(zt_flat, idx_flat):
    nt, o = zt_flat.shape
    info = plsc.get_sparse_core_info()
    nw = info.num_cores * info.num_subcores
    npt = nt // nw
    gpn = 4
    nslots = npt // gpn
    chunk = 128
    spc = chunk // gpn
    nch = npt // chunk
    nh = o // info.num_lanes
    nbuf = 4
    mesh = plsc.VectorSubcoreMesh(core_axis_name="c", subcore_axis_name="s")

    @functools.partial(
        pl.kernel,
        mesh=mesh,
        out_type=[jax.ShapeDtypeStruct((nt, o), jnp.float32)] * 3,
        scratch_types=[
            pltpu.VMEM((npt * K,), jnp.int32),
            pltpu.VMEM((nbuf, gpn * K, o), jnp.float32),
            pltpu.VMEM((chunk, o), jnp.float32),
            pltpu.VMEM((chunk, o), jnp.float32),
            pltpu.VMEM((chunk, o), jnp.float32),
            pltpu.SemaphoreType.DMA,
            pltpu.SemaphoreType.DMA,
            pltpu.SemaphoreType.DMA,
            pltpu.SemaphoreType.DMA,
        ],
    )
    def dkern(zt_hbm, idx_hbm, s1_hbm, s2_hbm, smax_hbm,
              idx_v, ring_v, b1, b2, b3, sem_a, sem_b, sem_c, sem_d):
        wid = lax.axis_index("s") * info.num_cores + lax.axis_index("c")
        base = wid * npt
        pltpu.sync_copy(idx_hbm.at[pl.ds(base * K, npt * K)], idx_v)
        sems = (sem_a, sem_b, sem_c, sem_d)

        def issue(slot, buf):
            pltpu.async_copy(
                zt_hbm.at[idx_v.at[pl.ds(slot * (gpn * K), gpn * K)]],
                ring_v.at[buf], sems[buf])

        def drain(buf):
            pltpu.make_async_copy(
                zt_hbm.at[pl.ds(0, gpn * K)], ring_v.at[buf], sems[buf]).wait()

        def consume(buf, i_local):
            def qbody(q, _):
                row0 = q * K
                for h in range(nh):
                    lane = pl.ds(h * info.num_lanes, info.num_lanes)
                    v0 = ring_v[buf, row0, lane]
                    a1, a2, am = v0, v0 * v0, v0
                    for t in range(1, K):
                        v = ring_v[buf, row0 + t, lane]
                        a1 = a1 + v
                        a2 = a2 + v * v
                        am = jnp.maximum(am, v)
                    b1[i_local + q, lane] = a1
                    b2[i_local + q, lane] = a2
                    b3[i_local + q, lane] = am
                return 0

            lax.fori_loop(0, gpn, qbody, 0)

        for bb in range(nbuf):
            issue(bb, bb)

        def chunk_body(ci, _):
            def pair_body(pi, _):
                for bb in range(nbuf):
                    si = pi * nbuf + bb
                    slot = ci * spc + si
                    drain(bb)
                    consume(bb, si * gpn)

                    @pl.when(slot + nbuf < nslots)
                    def _():
                        issue(slot + nbuf, bb)
                return 0

            lax.fori_loop(0, spc // nbuf, pair_body, 0)
            dst = pl.ds(base + ci * chunk, chunk)
            pltpu.sync_copy(b1, s1_hbm.at[dst])
            pltpu.sync_copy(b2, s2_hbm.at[dst])
            pltpu.sync_copy(b3, smax_hbm.at[dst])
            return 0

        lax.fori_loop(0, nch, chunk_body, 0)

    return dkern(zt_flat, idx_flat)


def _e1_body(s1_ref, s2_ref, y_ref, ssum_ref, qsum_ref):
    @pl.when(pl.program_id(0) == 0)
    def _():
        ssum_ref[...] = jnp.zeros_like(ssum_ref)
        qsum_ref[...] = jnp.zeros_like(qsum_ref)

    s1 = s1_ref[...]
    s2 = s2_ref[...]
    y = y_ref[...]
    kf = float(K)
    ssum_ref[...] += jnp.sum(s1 + kf * y, axis=0, keepdims=True)
    qsum_ref[...] += jnp.sum(s2 + 2.0 * y * s1 + kf * y * y, axis=0,
                             keepdims=True)


def _bn_stats(s1, s2, y_flat, rb=2048):
    nt, o = s1.shape
    return pl.pallas_call(
        _e1_body,
        grid=(nt // rb,),
        in_specs=[pl.BlockSpec((rb, o), lambda i: (i, 0))] * 3,
        out_specs=[pl.BlockSpec((1, o), lambda i: (0, 0))] * 2,
        out_shape=[jax.ShapeDtypeStruct((1, o), jnp.float32)] * 2,
    )(s1, s2, y_flat)


def _e2_body(smax_ref, y_ref, ssum_ref, qsum_ref, g_ref, bt_ref, o_ref, *,
             count):
    mean = ssum_ref[...] / count
    var = qsum_ref[...] / count - mean * mean
    scale = g_ref[...] * lax.rsqrt(var + EPS)
    t = (smax_ref[0] + y_ref[0] - mean) * scale + bt_ref[...]
    o_ref[0] = jnp.maximum(t, 0.0).T


def _finalize(smax, y, ssum, qsum, gamma, beta, rb=256):
    b, n, o = smax.shape
    count = float(b * n * K)
    return pl.pallas_call(
        functools.partial(_e2_body, count=count),
        grid=(b, n // rb),
        in_specs=[
            pl.BlockSpec((1, rb, o), lambda i, j: (i, j, 0)),
            pl.BlockSpec((1, rb, o), lambda i, j: (i, j, 0)),
            pl.BlockSpec((1, o), lambda i, j: (0, 0)),
            pl.BlockSpec((1, o), lambda i, j: (0, 0)),
            pl.BlockSpec((1, o), lambda i, j: (0, 0)),
            pl.BlockSpec((1, o), lambda i, j: (0, 0)),
        ],
        out_specs=pl.BlockSpec((1, o, rb), lambda i, j: (i, 0, j)),
        out_shape=jax.ShapeDtypeStruct((b, o, n), jnp.float32),
    )(smax, y, ssum, qsum, gamma.reshape(1, o), beta.reshape(1, o))


def kernel(x, W, gamma, beta):
    b, c, n = x.shape
    o = W.shape[0]
    w1 = W[:, :c]
    w1t = w1.T
    wst = (W[:, c:] - w1).T

    p, zt, yt = _pairwise_and_tables(x, w1t, wst)
    idx = _topk_sc(p.reshape(b * n, n))
    idx_flat = idx[:, :K].reshape(b * n * K)
    zt_flat = zt.reshape(b * n, o)
    s1, s2, smax = _gather_reduce_sc(zt_flat, idx_flat)
    y_flat = yt.reshape(b * n, o)
    ssum, qsum = _bn_stats(s1, s2, y_flat)
    return _finalize(smax.reshape(b, n, o), yt, ssum, qsum, gamma, beta)


# speedup vs baseline: 1.2116x; 1.2116x over previous
import functools

import jax
import jax.numpy as jnp
from jax import lax
from jax.experimental import pallas as pl
from jax.experimental.pallas import tpu as pltpu
from jax.experimental.pallas import tpu_sc as plsc

K = 20
EPS = 1e-5
NEG_INF = float("-inf")


def _a_body(xb_ref, xr_ref, w1t_ref, wst_ref, p_ref, zt_ref, yt_ref):
    xb = xb_ref[0]
    xr = xr_ref[0]
    dn = (((0,), (0,)), ((), ()))
    g = lax.dot_general(xr, xb, dn, preferred_element_type=jnp.float32)
    xx_col = jnp.sum(xb * xb, axis=0, keepdims=True)
    xx_row = jnp.sum(xr * xr, axis=0)[:, None]
    inner = -2.0 * g
    p_ref[0] = (-xx_col - inner) - xx_row
    zt_ref[0] = lax.dot_general(xr, w1t_ref[...], dn,
                                preferred_element_type=jnp.float32)
    yt_ref[0] = lax.dot_general(xr, wst_ref[...], dn,
                                preferred_element_type=jnp.float32)


def _pairwise_and_tables(x, w1t, wst, rb=256):
    b, c, n = x.shape
    o = w1t.shape[1]
    nrb = n // rb
    return pl.pallas_call(
        _a_body,
        grid=(b, nrb),
        in_specs=[
            pl.BlockSpec((1, c, n), lambda i, j: (i, 0, 0)),
            pl.BlockSpec((1, c, rb), lambda i, j: (i, 0, j)),
            pl.BlockSpec((c, o), lambda i, j: (0, 0)),
            pl.BlockSpec((c, o), lambda i, j: (0, 0)),
        ],
        out_specs=[
            pl.BlockSpec((1, rb, n), lambda i, j: (i, j, 0)),
            pl.BlockSpec((1, rb, o), lambda i, j: (i, j, 0)),
            pl.BlockSpec((1, rb, o), lambda i, j: (i, j, 0)),
        ],
        out_shape=[
            jax.ShapeDtypeStruct((b, n, n), jnp.float32),
            jax.ShapeDtypeStruct((b, n, o), jnp.float32),
            jax.ShapeDtypeStruct((b, n, o), jnp.float32),
        ],
    )(x, x, w1t, wst)


def _b_body(p_ref, idx_ref):
    bidx = pl.program_id(0)
    p = p_ref[0]
    r, n = p.shape
    iota = lax.broadcasted_iota(jnp.int32, (r, n), 1)
    tiota = lax.broadcasted_iota(jnp.int32, (r, 32), 1)
    off = bidx * n
    acc = jnp.zeros((r, 32), jnp.int32)
    for t in range(K):
        m = jnp.max(p, axis=1, keepdims=True)
        cand = jnp.where(p == m, iota, n)
        j = jnp.min(cand, axis=1, keepdims=True)
        acc = jnp.where(tiota == t, j + off, acc)
        p = jnp.where(iota == j, NEG_INF, p)
    idx_ref[0] = acc


def _topk_tc(p, rb=512):
    b, n, _ = p.shape
    return pl.pallas_call(
        _b_body,
        grid=(b, n // rb),
        in_specs=[pl.BlockSpec((1, rb, n), lambda i, j: (i, j, 0))],
        out_specs=pl.BlockSpec((1, rb, 32), lambda i, j: (i, j, 0)),
        out_shape=jax.ShapeDtypeStruct((b, n, 32), jnp.int32),
    )(p)


def _topk_sc(p_flat):
    nt, n = p_flat.shape
    info = plsc.get_sparse_core_info()
    L = info.num_lanes
    nw = info.num_cores * info.num_subcores
    rpw = nt // nw
    nv = n // L
    rpb = 4
    nbuf = 2
    och = 64
    nslots = rpw // rpb
    spf = och // rpb
    mesh = plsc.VectorSubcoreMesh(core_axis_name="c", subcore_axis_name="s")

    @functools.partial(
        pl.kernel,
        mesh=mesh,
        out_type=jax.ShapeDtypeStruct((nt, 32), jnp.int32),
        compiler_params=pltpu.CompilerParams(needs_layout_passes=False),
        scratch_types=[
            pltpu.VMEM((nbuf, rpb, n), jnp.float32),
            pltpu.VMEM((och, 32), jnp.int32),
            pltpu.SemaphoreType.DMA,
            pltpu.SemaphoreType.DMA,
        ],
    )
    def tkern(p_hbm, idx_hbm, ring_v, ob, sem_a, sem_b):
        wid = lax.axis_index("s") * info.num_cores + lax.axis_index("c")
        r0 = wid * rpw
        cbase = (r0 // n) * n
        sems = (sem_a, sem_b)
        iota = lax.iota(jnp.int32, L)

        def issue(slot, bb):
            pltpu.async_copy(p_hbm.at[pl.ds(r0 + slot * rpb, rpb)],
                             ring_v.at[bb], sems[bb])

        def drain(bb):
            pltpu.make_async_copy(p_hbm.at[pl.ds(0, rpb)], ring_v.at[bb],
                                  sems[bb]).wait()

        def bsplit(k0, v0, k1, v1):
            mx = k0 >= k1
            hk = jnp.where(mx, k0, k1)
            hv = jnp.where(mx, v0, v1)
            lk = jnp.where(mx, k1, k0)
            lv = jnp.where(mx, v1, v0)
            hk, hv = plsc.sort_key_val(hk, hv, descending=True)
            lk, lv = plsc.sort_key_val(lk, lv, descending=True)
            return ((hk, hv), (lk, lv))

        def leaf(k0, v0, k1, v1):
            k0, v0 = plsc.sort_key_val(k0, v0, descending=True)
            k1, v1 = plsc.sort_key_val(k1, v1, descending=False)
            return bsplit(k0, v0, k1, v1)

        def merge32(a, b):
            (ak0, av0), (ak1, av1) = a
            (bk0, bv0), (bk1, bv1) = b
            rk0 = lax.rev(bk1, (0,))
            rv0 = lax.rev(bv1, (0,))
            rk1 = lax.rev(bk0, (0,))
            rv1 = lax.rev(bv0, (0,))
            m0 = ak0 >= rk0
            c0k = jnp.where(m0, ak0, rk0)
            c0v = jnp.where(m0, av0, rv0)
            m1 = ak1 >= rk1
            c1k = jnp.where(m1, ak1, rk1)
            c1v = jnp.where(m1, av1, rv1)
            return bsplit(c0k, c0v, c1k, c1v)

        ninf = jnp.full((L,), NEG_INF, jnp.float32)
        zero = jnp.zeros((L,), jnp.int32)

        def store_run(run, rl):
            (_, hv), (_, lv) = run
            ob[rl, pl.ds(0, L)] = hv + cbase
            ob[rl, pl.ds(L, L)] = lv + cbase

        def row_topk(bb, ri, rl):
            def load(v):
                return ring_v[bb, ri, pl.ds(v * L, L)]

            grp = 8
            run = None
            for g in range(nv // 2 // grp):
                nodes = []
                for e in range(grp):
                    v2 = (g * grp + e) * 2
                    nodes.append(leaf(load(v2), iota + v2 * L,
                                      load(v2 + 1), iota + (v2 + 1) * L))
                while len(nodes) > 1:
                    nodes = [merge32(nodes[i], nodes[i + 1])
                             for i in range(0, len(nodes), 2)]
                run = nodes[0] if run is None else merge32(run, nodes[0])
            store_run(run, rl)

        for bb in range(nbuf):
            issue(bb, bb)

        def flush_body(fc, _):
            def pair_body(sp, _):
                for bb in range(nbuf):
                    si = sp * nbuf + bb
                    slot = fc * spf + si
                    drain(bb)

                    def rbody(ri, _):
                        row_topk(bb, ri, si * rpb + ri)
                        return 0

                    lax.fori_loop(0, rpb, rbody, 0)

                    @pl.when(slot + nbuf < nslots)
                    def _():
                        issue(slot + nbuf, bb)
                return 0

            lax.fori_loop(0, spf // nbuf, pair_body, 0)
            pltpu.sync_copy(ob, idx_hbm.at[pl.ds(r0 + fc * och, och)])
            return 0

        lax.fori_loop(0, nslots // spf, flush_body, 0)

    return tkern(p_flat)


def _gather_reduce_sc(zt_flat, idx_flat):
    nt, o = zt_flat.shape
    info = plsc.get_sparse_core_info()
    nw = info.num_cores * info.num_subcores
    npt = nt // nw
    gpn = 4
    nslots = npt // gpn
    chunk = 128
    spc = chunk // gpn
    nch = npt // chunk
    nh = o // info.num_lanes
    nbuf = 4
    mesh = plsc.VectorSubcoreMesh(core_axis_name="c", subcore_axis_name="s")

    @functools.partial(
        pl.kernel,
        mesh=mesh,
        out_type=[jax.ShapeDtypeStruct((nt, o), jnp.float32)] * 3,
        scratch_types=[
            pltpu.VMEM((npt * K,), jnp.int32),
            pltpu.VMEM((nbuf, gpn * K, o), jnp.float32),
            pltpu.VMEM((chunk, o), jnp.float32),
            pltpu.VMEM((chunk, o), jnp.float32),
            pltpu.VMEM((chunk, o), jnp.float32),
            pltpu.SemaphoreType.DMA,
            pltpu.SemaphoreType.DMA,
            pltpu.SemaphoreType.DMA,
            pltpu.SemaphoreType.DMA,
        ],
    )
    def dkern(zt_hbm, idx_hbm, s1_hbm, s2_hbm, smax_hbm,
              idx_v, ring_v, b1, b2, b3, sem_a, sem_b, sem_c, sem_d):
        wid = lax.axis_index("s") * info.num_cores + lax.axis_index("c")
        base = wid * npt
        pltpu.sync_copy(idx_hbm.at[pl.ds(base * K, npt * K)], idx_v)
        sems = (sem_a, sem_b, sem_c, sem_d)

        def issue(slot, buf):
            pltpu.async_copy(
                zt_hbm.at[idx_v.at[pl.ds(slot * (gpn * K), gpn * K)]],
                ring_v.at[buf], sems[buf])

        def drain(buf):
            pltpu.make_async_copy(
                zt_hbm.at[pl.ds(0, gpn * K)], ring_v.at[buf], sems[buf]).wait()

        def consume(buf, i_local):
            def qbody(q, _):
                row0 = q * K
                for h in range(nh):
                    lane = pl.ds(h * info.num_lanes, info.num_lanes)
                    v0 = ring_v[buf, row0, lane]
                    a1, a2, am = v0, v0 * v0, v0
                    for t in range(1, K):
                        v = ring_v[buf, row0 + t, lane]
                        a1 = a1 + v
                        a2 = a2 + v * v
                        am = jnp.maximum(am, v)
                    b1[i_local + q, lane] = a1
                    b2[i_local + q, lane] = a2
                    b3[i_local + q, lane] = am
                return 0

            lax.fori_loop(0, gpn, qbody, 0)

        for bb in range(nbuf):
            issue(bb, bb)

        def chunk_body(ci, _):
            def pair_body(pi, _):
                for bb in range(nbuf):
                    si = pi * nbuf + bb
                    slot = ci * spc + si
                    drain(bb)
                    consume(bb, si * gpn)

                    @pl.when(slot + nbuf < nslots)
                    def _():
                        issue(slot + nbuf, bb)
                return 0

            lax.fori_loop(0, spc // nbuf, pair_body, 0)
            dst = pl.ds(base + ci * chunk, chunk)
            pltpu.sync_copy(b1, s1_hbm.at[dst])
            pltpu.sync_copy(b2, s2_hbm.at[dst])
            pltpu.sync_copy(b3, smax_hbm.at[dst])
            return 0

        lax.fori_loop(0, nch, chunk_body, 0)

    return dkern(zt_flat, idx_flat)


def _e1_body(s1_ref, s2_ref, y_ref, ssum_ref, qsum_ref):
    @pl.when(pl.program_id(0) == 0)
    def _():
        ssum_ref[...] = jnp.zeros_like(ssum_ref)
        qsum_ref[...] = jnp.zeros_like(qsum_ref)

    s1 = s1_ref[...]
    s2 = s2_ref[...]
    y = y_ref[...]
    kf = float(K)
    ssum_ref[...] += jnp.sum(s1 + kf * y, axis=0, keepdims=True)
    qsum_ref[...] += jnp.sum(s2 + 2.0 * y * s1 + kf * y * y, axis=0,
                             keepdims=True)


def _bn_stats(s1, s2, y_flat, rb=2048):
    nt, o = s1.shape
    return pl.pallas_call(
        _e1_body,
        grid=(nt // rb,),
        in_specs=[pl.BlockSpec((rb, o), lambda i: (i, 0))] * 3,
        out_specs=[pl.BlockSpec((1, o), lambda i: (0, 0))] * 2,
        out_shape=[jax.ShapeDtypeStruct((1, o), jnp.float32)] * 2,
    )(s1, s2, y_flat)


def _e2_body(smax_ref, y_ref, ssum_ref, qsum_ref, g_ref, bt_ref, o_ref, *,
             count):
    mean = ssum_ref[...] / count
    var = qsum_ref[...] / count - mean * mean
    scale = g_ref[...] * lax.rsqrt(var + EPS)
    t = (smax_ref[0] + y_ref[0] - mean) * scale + bt_ref[...]
    o_ref[0] = jnp.maximum(t, 0.0).T


def _finalize(smax, y, ssum, qsum, gamma, beta, rb=256):
    b, n, o = smax.shape
    count = float(b * n * K)
    return pl.pallas_call(
        functools.partial(_e2_body, count=count),
        grid=(b, n // rb),
        in_specs=[
            pl.BlockSpec((1, rb, o), lambda i, j: (i, j, 0)),
            pl.BlockSpec((1, rb, o), lambda i, j: (i, j, 0)),
            pl.BlockSpec((1, o), lambda i, j: (0, 0)),
            pl.BlockSpec((1, o), lambda i, j: (0, 0)),
            pl.BlockSpec((1, o), lambda i, j: (0, 0)),
            pl.BlockSpec((1, o), lambda i, j: (0, 0)),
        ],
        out_specs=pl.BlockSpec((1, o, rb), lambda i, j: (i, 0, j)),
        out_shape=jax.ShapeDtypeStruct((b, o, n), jnp.float32),
    )(smax, y, ssum, qsum, gamma.reshape(1, o), beta.reshape(1, o))


def kernel(x, W, gamma, beta):
    b, c, n = x.shape
    o = W.shape[0]
    w1 = W[:, :c]
    w1t = w1.T
    wst = (W[:, c:] - w1).T

    p, zt, yt = _pairwise_and_tables(x, w1t, wst)
    idx = _topk_sc(p.reshape(b * n, n))
    idx_flat = idx[:, :K].reshape(b * n * K)
    zt_flat = zt.reshape(b * n, o)
    s1, s2, smax = _gather_reduce_sc(zt_flat, idx_flat)
    y_flat = yt.reshape(b * n, o)
    ssum, qsum = _bn_stats(s1, s2, y_flat)
    return _finalize(smax.reshape(b, n, o), yt, ssum, qsum, gamma, beta)
